# Initial kernel scaffold; baseline (speedup 1.0000x reference)
import jax
import jax.numpy as jnp
from jax.experimental import pallas as pl


def kernel(events_x, events_y, calib_center, precompute_grid, pairwise_dists_mask, pairwise_dists, correction):
    raise NotImplementedError("write your pallas kernel here")


# speedup vs baseline: 481.9394x; 481.9394x over previous
import functools

import jax
import jax.numpy as jnp
from jax import lax
from jax.experimental import pallas as pl
from jax.experimental.pallas import tpu as pltpu
from jax.experimental.pallas import tpu_sc as plsc

D = 256
E = 16384
G = 101
TAB = G * G
TABP = TAB + 7
NC = 2
NS = 16
NW = NC * NS
DPW = D // NW
L = 16
CHUNKS = E // L
RADIUS = 50


def _sc_body(fex_hbm, fey_hbm, t0_hbm, t1_hbm, ct_hbm, calib_hbm,
             sum0_hbm, sum1_hbm, cnt_hbm,
             fex_v, fey_v, t0_v, t1_v, ct_v, calib_v, o0_v, o1_v, oc_v):
    c = lax.axis_index("c")
    s = lax.axis_index("s")
    wid = s * NC + c

    pltpu.sync_copy(fex_hbm, fex_v)
    pltpu.sync_copy(fey_hbm, fey_v)
    pltpu.sync_copy(t0_hbm, t0_v)
    pltpu.sync_copy(t1_hbm, t1_v)
    pltpu.sync_copy(ct_hbm, ct_v)
    pltpu.sync_copy(calib_hbm.at[pl.ds(wid * (2 * DPW), 2 * DPW)], calib_v)

    cvec = calib_v[...]
    cxs = []
    cys = []
    for d in range(DPW):
        cxs.append(jnp.full((L,), cvec[2 * d + 1], jnp.float32))
        cys.append(jnp.full((L,), cvec[2 * d], jnp.float32))

    zf = jnp.zeros((L,), jnp.float32)
    zi = jnp.zeros((L,), jnp.int32)
    init = tuple([zf] * DPW + [zf] * DPW + [zi] * DPW)

    def body(i, carry):
        accs = list(carry)
        ex = fex_v[pl.ds(i * L, L)]
        ey = fey_v[pl.ds(i * L, L)]
        for d in range(DPW):
            dx = (ex - cxs[d]).astype(jnp.int32)
            dy = (ey - cys[d]).astype(jnp.int32)
            ix = jnp.minimum(jnp.maximum(dx, -RADIUS), RADIUS)
            iy = jnp.minimum(jnp.maximum(dy, -RADIUS), RADIUS)
            flat = ix * G + (iy + (RADIUS * G + RADIUS))
            g0 = plsc.load_gather(t0_v, [flat])
            g1 = plsc.load_gather(t1_v, [flat])
            gc = plsc.load_gather(ct_v, [flat])
            accs[d] = accs[d] + g0
            accs[DPW + d] = accs[DPW + d] + g1
            accs[2 * DPW + d] = accs[2 * DPW + d] + gc
        return tuple(accs)

    accs = lax.fori_loop(0, CHUNKS, body, init)

    lanes = lax.iota(jnp.int32, L)
    o0 = zf
    o1 = zf
    oc = zi
    for d in range(DPW):
        o0 = jnp.where(lanes == d, jnp.sum(accs[d]), o0)
        o1 = jnp.where(lanes == d, jnp.sum(accs[DPW + d]), o1)
        oc = jnp.where(lanes == d, jnp.sum(accs[2 * DPW + d]), oc)
    o0_v[...] = o0
    o1_v[...] = o1
    oc_v[...] = oc
    pltpu.sync_copy(o0_v, sum0_hbm.at[pl.ds(wid * L, L)])
    pltpu.sync_copy(o1_v, sum1_hbm.at[pl.ds(wid * L, L)])
    pltpu.sync_copy(oc_v, cnt_hbm.at[pl.ds(wid * L, L)])


_sc_gather = pl.kernel(
    _sc_body,
    out_type=[
        jax.ShapeDtypeStruct((NW * L,), jnp.float32),
        jax.ShapeDtypeStruct((NW * L,), jnp.float32),
        jax.ShapeDtypeStruct((NW * L,), jnp.int32),
    ],
    mesh=plsc.VectorSubcoreMesh(core_axis_name="c", subcore_axis_name="s"),
    compiler_params=pltpu.CompilerParams(needs_layout_passes=False),
    scratch_types=[
        pltpu.VMEM((E,), jnp.float32),
        pltpu.VMEM((E,), jnp.float32),
        pltpu.VMEM((TABP,), jnp.float32),
        pltpu.VMEM((TABP,), jnp.float32),
        pltpu.VMEM((TABP,), jnp.int32),
        pltpu.VMEM((2 * DPW,), jnp.float32),
        pltpu.VMEM((L,), jnp.float32),
        pltpu.VMEM((L,), jnp.float32),
        pltpu.VMEM((L,), jnp.int32),
    ],
)


def _tc_body(c0c_ref, c1c_ref, c0r_ref, c1r_ref, m_ref, pd_ref,
             s0_ref, s1_ref, cnt_ref, corr_ref,
             new0_ref, new1_ref, ne_ref):
    c0c = c0c_ref[...]
    c1c = c1c_ref[...]
    dxc = c1r_ref[...] - c1c
    dyc = c0r_ref[...] - c0c
    m = m_ref[...]
    pd = pd_ref[...]
    sdx = dxc * m
    sdy = dyc * m
    radi = sdx * sdx + sdy * sdy - pd * pd
    corr = corr_ref[0, 0]
    cdx = corr * jnp.sum(4.0 * dxc * radi, axis=1, keepdims=True)
    cdy = corr * jnp.sum(4.0 * dyc * radi, axis=1, keepdims=True)
    cnt = cnt_ref[...]
    dec = ((cnt & 0xFFFF) >= 10).astype(jnp.float32)
    ne_ref[...] = lax.shift_right_arithmetic(cnt, 16)
    lr = jnp.float32(200 * 1.5e-05)
    rf = jnp.float32(1.0 * 2.5e-07)
    new1_ref[...] = c1c - lr * dec * (
        jnp.clip(s0_ref[...], -400.0, 400.0) - rf * cdx)
    new0_ref[...] = c0c - lr * dec * (
        jnp.clip(s1_ref[...], -400.0, 400.0) - rf * cdy)


_tc_reg = pl.pallas_call(
    _tc_body,
    out_shape=[
        jax.ShapeDtypeStruct((D, 1), jnp.float32),
        jax.ShapeDtypeStruct((D, 1), jnp.float32),
        jax.ShapeDtypeStruct((D, 1), jnp.int32),
    ],
)


def kernel(events_x, events_y, calib_center, precompute_grid,
           pairwise_dists_mask, pairwise_dists, correction):
    fex = events_x.astype(jnp.float32)
    fey = events_y.astype(jnp.float32)
    t0 = precompute_grid[:, :, 0].reshape(-1)
    t1 = precompute_grid[:, :, 1].reshape(-1)
    nz = (t0 != 0).astype(jnp.int32) + (t1 != 0).astype(jnp.int32)
    ii = jnp.arange(G, dtype=jnp.int32)
    interior = jnp.logical_and(ii >= 1, ii <= G - 2)
    vic = jnp.logical_and(interior[:, None], interior[None, :])
    ct = nz + (vic.reshape(-1).astype(jnp.int32) << 16)
    t0p = jnp.pad(t0, (0, TABP - TAB))
    t1p = jnp.pad(t1, (0, TABP - TAB))
    ctp = jnp.pad(ct, (0, TABP - TAB))

    sum0, sum1, cnt = _sc_gather(fex, fey, t0p, t1p, ctp,
                                 calib_center.reshape(2 * D))
    sum0 = sum0.reshape(NW, L)[:, :DPW].reshape(D, 1)
    sum1 = sum1.reshape(NW, L)[:, :DPW].reshape(D, 1)
    cnt = cnt.reshape(NW, L)[:, :DPW].reshape(D, 1)

    c0c = calib_center[:, 0].reshape(D, 1)
    c1c = calib_center[:, 1].reshape(D, 1)
    c0r = calib_center[:, 0].reshape(1, D)
    c1r = calib_center[:, 1].reshape(1, D)
    corr = correction.reshape(1, 1).astype(jnp.float32)

    new0, new1, ne = _tc_reg(c0c, c1c, c0r, c1r, pairwise_dists_mask,
                             pairwise_dists, sum0, sum1, cnt, corr)
    calib_out = jnp.concatenate([new0, new1], axis=1)
    return (calib_out, ne.reshape(D))


# speedup vs baseline: 536.3457x; 1.1129x over previous
import functools

import jax
import jax.numpy as jnp
from jax import lax
from jax.experimental import pallas as pl
from jax.experimental.pallas import tpu as pltpu
from jax.experimental.pallas import tpu_sc as plsc

D = 256
E = 16384
G = 101
TAB = G * G
TABP = TAB + 7
NC = 2
NS = 16
NW = NC * NS
DPW = D // NW
L = 16
CHUNKS = E // L
RADIUS = 50


def _sc_body(fex_hbm, fey_hbm, t0_hbm, t1_hbm, ct_hbm, calib_hbm,
             sum0_hbm, sum1_hbm, cnt_hbm,
             fex_v, fey_v, t0_v, t1_v, ct_v, calib_v, o0_v, o1_v, oc_v,
             sem):
    c = lax.axis_index("c")
    s = lax.axis_index("s")
    wid = s * NC + c

    cp = []
    cp.append(pltpu.async_copy(fex_hbm, fex_v, sem))
    cp.append(pltpu.async_copy(fey_hbm, fey_v, sem))
    cp.append(pltpu.async_copy(t0_hbm, t0_v, sem))
    cp.append(pltpu.async_copy(t1_hbm, t1_v, sem))
    cp.append(pltpu.async_copy(ct_hbm, ct_v, sem))
    cp.append(pltpu.async_copy(
        calib_hbm.at[pl.ds(wid * (2 * DPW), 2 * DPW)], calib_v, sem))
    for h in cp:
        h.wait()

    cvec = calib_v[...]
    cxs = []
    cys = []
    for d in range(DPW):
        cxs.append(jnp.full((L,), cvec[2 * d + 1], jnp.float32))
        cys.append(jnp.full((L,), cvec[2 * d], jnp.float32))

    zf = jnp.zeros((L,), jnp.float32)
    zi = jnp.zeros((L,), jnp.int32)
    init = tuple([zf] * DPW + [zf] * DPW + [zi] * DPW)

    def body(i, carry):
        accs = list(carry)
        ex = fex_v[pl.ds(i * L, L)]
        ey = fey_v[pl.ds(i * L, L)]
        for d in range(DPW):
            fr = jnp.float32(RADIUS)
            ix = jnp.clip(ex - cxs[d], -fr, fr).astype(jnp.int32)
            iy = jnp.clip(ey - cys[d], -fr, fr).astype(jnp.int32)
            flat = ix * G + (iy + (RADIUS * G + RADIUS))
            g0 = plsc.load_gather(t0_v, [flat])
            g1 = plsc.load_gather(t1_v, [flat])
            gc = plsc.load_gather(ct_v, [flat])
            accs[d] = accs[d] + g0
            accs[DPW + d] = accs[DPW + d] + g1
            accs[2 * DPW + d] = accs[2 * DPW + d] + gc
        return tuple(accs)

    accs = lax.fori_loop(0, CHUNKS, body, init)

    lanes = lax.iota(jnp.int32, L)
    o0 = zf
    o1 = zf
    oc = zi
    for d in range(DPW):
        o0 = jnp.where(lanes == d, jnp.sum(accs[d]), o0)
        o1 = jnp.where(lanes == d, jnp.sum(accs[DPW + d]), o1)
        oc = jnp.where(lanes == d, jnp.sum(accs[2 * DPW + d]), oc)
    o0_v[...] = o0
    o1_v[...] = o1
    oc_v[...] = oc
    pltpu.sync_copy(o0_v, sum0_hbm.at[pl.ds(wid * L, L)])
    pltpu.sync_copy(o1_v, sum1_hbm.at[pl.ds(wid * L, L)])
    pltpu.sync_copy(oc_v, cnt_hbm.at[pl.ds(wid * L, L)])


_sc_gather = pl.kernel(
    _sc_body,
    out_type=[
        jax.ShapeDtypeStruct((NW * L,), jnp.float32),
        jax.ShapeDtypeStruct((NW * L,), jnp.float32),
        jax.ShapeDtypeStruct((NW * L,), jnp.int32),
    ],
    mesh=plsc.VectorSubcoreMesh(core_axis_name="c", subcore_axis_name="s"),
    compiler_params=pltpu.CompilerParams(needs_layout_passes=False),
    scratch_types=[
        pltpu.VMEM((E,), jnp.float32),
        pltpu.VMEM((E,), jnp.float32),
        pltpu.VMEM((TABP,), jnp.float32),
        pltpu.VMEM((TABP,), jnp.float32),
        pltpu.VMEM((TABP,), jnp.int32),
        pltpu.VMEM((2 * DPW,), jnp.float32),
        pltpu.VMEM((L,), jnp.float32),
        pltpu.VMEM((L,), jnp.float32),
        pltpu.VMEM((L,), jnp.int32),
        pltpu.SemaphoreType.DMA,
    ],
)


def _tc_body(c0c_ref, c1c_ref, c0r_ref, c1r_ref, m_ref, pd_ref,
             s0_ref, s1_ref, cnt_ref, corr_ref,
             new0_ref, new1_ref, ne_ref):
    c0c = c0c_ref[...]
    c1c = c1c_ref[...]
    dxc = c1r_ref[...] - c1c
    dyc = c0r_ref[...] - c0c
    m = m_ref[...]
    pd = pd_ref[...]
    sdx = dxc * m
    sdy = dyc * m
    radi = sdx * sdx + sdy * sdy - pd * pd
    corr = corr_ref[0, 0]
    cdx = corr * jnp.sum(4.0 * dxc * radi, axis=1, keepdims=True)
    cdy = corr * jnp.sum(4.0 * dyc * radi, axis=1, keepdims=True)
    cnt = cnt_ref[...]
    dec = ((cnt & 0xFFFF) >= 10).astype(jnp.float32)
    ne_ref[...] = lax.shift_right_arithmetic(cnt, 16)
    lr = jnp.float32(200 * 1.5e-05)
    rf = jnp.float32(1.0 * 2.5e-07)
    new1_ref[...] = c1c - lr * dec * (
        jnp.clip(s0_ref[...], -400.0, 400.0) - rf * cdx)
    new0_ref[...] = c0c - lr * dec * (
        jnp.clip(s1_ref[...], -400.0, 400.0) - rf * cdy)


_tc_reg = pl.pallas_call(
    _tc_body,
    out_shape=[
        jax.ShapeDtypeStruct((D, 1), jnp.float32),
        jax.ShapeDtypeStruct((D, 1), jnp.float32),
        jax.ShapeDtypeStruct((D, 1), jnp.int32),
    ],
)


def kernel(events_x, events_y, calib_center, precompute_grid,
           pairwise_dists_mask, pairwise_dists, correction):
    fex = events_x.astype(jnp.float32)
    fey = events_y.astype(jnp.float32)
    t0 = precompute_grid[:, :, 0].reshape(-1)
    t1 = precompute_grid[:, :, 1].reshape(-1)
    nz = (t0 != 0).astype(jnp.int32) + (t1 != 0).astype(jnp.int32)
    ii = jnp.arange(G, dtype=jnp.int32)
    interior = jnp.logical_and(ii >= 1, ii <= G - 2)
    vic = jnp.logical_and(interior[:, None], interior[None, :])
    ct = nz + (vic.reshape(-1).astype(jnp.int32) << 16)
    t0p = jnp.pad(t0, (0, TABP - TAB))
    t1p = jnp.pad(t1, (0, TABP - TAB))
    ctp = jnp.pad(ct, (0, TABP - TAB))

    sum0, sum1, cnt = _sc_gather(fex, fey, t0p, t1p, ctp,
                                 calib_center.reshape(2 * D))
    sum0 = sum0.reshape(NW, L)[:, :DPW].reshape(D, 1)
    sum1 = sum1.reshape(NW, L)[:, :DPW].reshape(D, 1)
    cnt = cnt.reshape(NW, L)[:, :DPW].reshape(D, 1)

    c0c = calib_center[:, 0].reshape(D, 1)
    c1c = calib_center[:, 1].reshape(D, 1)
    c0r = calib_center[:, 0].reshape(1, D)
    c1r = calib_center[:, 1].reshape(1, D)
    corr = correction.reshape(1, 1).astype(jnp.float32)

    new0, new1, ne = _tc_reg(c0c, c1c, c0r, c1r, pairwise_dists_mask,
                             pairwise_dists, sum0, sum1, cnt, corr)
    calib_out = jnp.concatenate([new0, new1], axis=1)
    return (calib_out, ne.reshape(D))
